# Initial kernel scaffold; baseline (speedup 1.0000x reference)
import functools

import jax
import jax.numpy as jnp
from jax import lax
from jax.experimental import pallas as pl
from jax.experimental.pallas import tpu as pltpu
from jax.experimental.pallas import tpu_sc as plsc

VOCAB = 100000
MAXLEN = 60
D = 144
BATCH = 16384
SEQ = 50
NV = D // 16

NC, NS = 2, 16
NW = NC * NS
ROWS = BATCH * SEQ
ROWS_PER_W = ROWS // NW
CHUNK = 100
NCHUNK = ROWS_PER_W // CHUNK


def _vrsqrt(x):
    i = plsc.bitcast(x, jnp.int32)
    y = plsc.bitcast(jnp.int32(0x5F3759DF) - (i >> 1), jnp.float32)
    for _ in range(3):
        y = y * (1.5 - 0.5 * x * y * y)
    return y


def _sc_body(x_hbm, tok_hbm, pos_hbm, gamma_hbm, beta_hbm, out_hbm,
             idx_v, in_v, out_v, pos_v, gb_v, sem):
    wid = lax.axis_index("s") * NC + lax.axis_index("c")
    base = wid * ROWS_PER_W

    pltpu.sync_copy(x_hbm.at[pl.ds(base, ROWS_PER_W)], idx_v)
    pltpu.sync_copy(pos_hbm.at[pl.ds(0, SEQ)], pos_v)
    pltpu.sync_copy(gamma_hbm, gb_v.at[0])
    pltpu.sync_copy(beta_hbm, gb_v.at[1])

    g_vecs = [gb_v[0, pl.ds(k * 16, 16)] for k in range(NV)]
    b_vecs = [gb_v[1, pl.ds(k * 16, 16)] for k in range(NV)]

    @pl.loop(0, NCHUNK)
    def chunk_loop(g):
        pltpu.async_copy(
            tok_hbm.at[idx_v.at[pl.ds(g * CHUNK, CHUNK)]], in_v, sem
        ).wait()

        @pl.loop(0, CHUNK)
        def row_loop(j):
            p = lax.rem(j, SEQ)
            vs = [in_v[j, pl.ds(k * 16, 16)] + pos_v[p, pl.ds(k * 16, 16)]
                  for k in range(NV)]
            s = vs[0]
            for k in range(1, NV):
                s = s + vs[k]
            mean = jnp.sum(s) * (1.0 / D)
            cs = [v - mean for v in vs]
            sq = cs[0] * cs[0]
            for k in range(1, NV):
                sq = sq + cs[k] * cs[k]
            var = jnp.sum(sq) * (1.0 / D)
            r = _vrsqrt(jnp.broadcast_to(var + 1e-5, (16,)))
            for k in range(NV):
                out_v[j, pl.ds(k * 16, 16)] = cs[k] * (r * g_vecs[k]) + b_vecs[k]

        pltpu.sync_copy(out_v, out_hbm.at[pl.ds(base + g * CHUNK, CHUNK)])


@jax.jit
def kernel(x, tok_table, pos_table, gamma, beta):
    x_flat = x.reshape(-1).astype(jnp.int32)
    mesh = plsc.VectorSubcoreMesh(core_axis_name="c", subcore_axis_name="s")
    out = pl.kernel(
        _sc_body,
        out_type=jax.ShapeDtypeStruct((ROWS, D), jnp.float32),
        mesh=mesh,
        scratch_types=[
            pltpu.VMEM((ROWS_PER_W,), jnp.int32),
            pltpu.VMEM((CHUNK, D), jnp.float32),
            pltpu.VMEM((CHUNK, D), jnp.float32),
            pltpu.VMEM((SEQ, D), jnp.float32),
            pltpu.VMEM((2, D), jnp.float32),
            pltpu.SemaphoreType.DMA,
        ],
    )(x_flat, tok_table, pos_table, gamma, beta)
    return out.reshape(BATCH, SEQ, D)


# speedup vs baseline: 5.0018x; 5.0018x over previous
import functools

import jax
import jax.numpy as jnp
from jax import lax
from jax.experimental import pallas as pl
from jax.experimental.pallas import tpu as pltpu
from jax.experimental.pallas import tpu_sc as plsc

VOCAB = 100000
MAXLEN = 60
D = 144
BATCH = 16384
SEQ = 50
NV = D // 16

NC, NS = 2, 16
NW = NC * NS
ROWS = BATCH * SEQ
ROWS_PER_W = ROWS // NW
CHUNK = 200
NCHUNK = ROWS_PER_W // CHUNK


def _hsum(v, lanes):
    for s in (8, 4, 2, 1):
        v = v + v.at[lanes ^ s].get(mode="promise_in_bounds")
    return v


def _vrsqrt(x):
    i = lax.bitcast_convert_type(x, jnp.int32)
    y = lax.bitcast_convert_type(jnp.int32(0x5F3759DF) - (i >> 1), jnp.float32)
    for _ in range(3):
        y = y * (1.5 - 0.5 * x * y * y)
    return y


def _sc_body(x_hbm, tok_hbm, pos_hbm, gamma_hbm, beta_hbm, out_hbm,
             idx_v, in_v, out_v, pos_v, gb_v, sem):
    wid = lax.axis_index("s") * NC + lax.axis_index("c")
    base = wid * ROWS_PER_W

    pltpu.sync_copy(x_hbm.at[pl.ds(base, ROWS_PER_W)], idx_v)
    pltpu.sync_copy(pos_hbm, pos_v)
    pltpu.sync_copy(gamma_hbm, gb_v.at[0])
    pltpu.sync_copy(beta_hbm, gb_v.at[1])

    g_vecs = [gb_v[0, pl.ds(k * 16, 16)] for k in range(NV)]
    b_vecs = [gb_v[1, pl.ds(k * 16, 16)] for k in range(NV)]
    lanes = lax.iota(jnp.int32, 16)

    @pl.loop(0, NCHUNK)
    def chunk_loop(g):
        pltpu.async_copy(
            tok_hbm.at[idx_v.at[pl.ds(g * CHUNK, CHUNK)]], in_v, sem
        ).wait()

        @pl.loop(0, CHUNK)
        def row_loop(j):
            p = lax.rem(j, SEQ)
            vs = [in_v[j, pl.ds(k * 16, 16)] + pos_v[p, pl.ds(k * 16, 16)]
                  for k in range(NV)]
            s = vs[0]
            for k in range(1, NV):
                s = s + vs[k]
            mean = _hsum(s, lanes) * (1.0 / D)
            cs = [v - mean for v in vs]
            sq = cs[0] * cs[0]
            for k in range(1, NV):
                sq = sq + cs[k] * cs[k]
            var = _hsum(sq, lanes) * (1.0 / D)
            r = _vrsqrt(var + 1e-5)
            for k in range(NV):
                out_v[j, pl.ds(k * 16, 16)] = cs[k] * (r * g_vecs[k]) + b_vecs[k]

        pltpu.sync_copy(out_v, out_hbm.at[pl.ds(base + g * CHUNK, CHUNK)])


@jax.jit
def kernel(x, tok_table, pos_table, gamma, beta):
    x_flat = x.reshape(-1).astype(jnp.int32)
    mesh = plsc.VectorSubcoreMesh(core_axis_name="c", subcore_axis_name="s")
    out = pl.kernel(
        _sc_body,
        out_type=jax.ShapeDtypeStruct((ROWS, D), jnp.float32),
        mesh=mesh,
        compiler_params=pltpu.CompilerParams(use_tc_tiling_on_sc=False),
        scratch_types=[
            pltpu.VMEM((ROWS_PER_W,), jnp.int32),
            pltpu.VMEM((CHUNK, D), jnp.float32),
            pltpu.VMEM((CHUNK, D), jnp.float32),
            pltpu.VMEM((MAXLEN, D), jnp.float32),
            pltpu.VMEM((2, D), jnp.float32),
            pltpu.SemaphoreType.DMA,
        ],
    )(x_flat, tok_table, pos_table, gamma, beta)
    return out.reshape(BATCH, SEQ, D)


# speedup vs baseline: 6.4950x; 1.2985x over previous
import functools

import jax
import jax.numpy as jnp
from jax import lax
from jax.experimental import pallas as pl
from jax.experimental.pallas import tpu as pltpu
from jax.experimental.pallas import tpu_sc as plsc

VOCAB = 100000
MAXLEN = 60
D = 144
BATCH = 16384
SEQ = 50
NV = D // 16

NC, NS = 2, 16
NW = NC * NS
ROWS = BATCH * SEQ
ROWS_PER_W = ROWS // NW
CHUNK = 200
NCHUNK = ROWS_PER_W // CHUNK


def _hsum(v, lanes):
    for s in (8, 4, 2, 1):
        v = v + v.at[lanes ^ s].get(mode="promise_in_bounds")
    return v


def _vrsqrt(x):
    i = lax.bitcast_convert_type(x, jnp.int32)
    y = lax.bitcast_convert_type(jnp.int32(0x5F3759DF) - (i >> 1), jnp.float32)
    for _ in range(2):
        y = y * (1.5 - 0.5 * x * y * y)
    return y


def _tree_sum(vs):
    while len(vs) > 1:
        vs = [vs[i] + vs[i + 1] for i in range(0, len(vs) - 1, 2)] + (
            [vs[-1]] if len(vs) % 2 else [])
    return vs[0]


def _sc_body(x_hbm, tok_hbm, pos_hbm, gamma_hbm, beta_hbm, out_hbm,
             idx_v, in0_v, in1_v, out_v, pos_v, gb_v,
             gsem0, gsem1, ssem):
    wid = lax.axis_index("s") * NC + lax.axis_index("c")
    base = wid * ROWS_PER_W
    in_bufs = (in0_v, in1_v)
    gsems = (gsem0, gsem1)

    pltpu.sync_copy(x_hbm.at[pl.ds(base, ROWS_PER_W)], idx_v)
    pltpu.sync_copy(pos_hbm, pos_v)
    pltpu.sync_copy(gamma_hbm, gb_v.at[0])
    pltpu.sync_copy(beta_hbm, gb_v.at[1])

    g_vecs = [gb_v[0, pl.ds(k * 16, 16)] for k in range(NV)]
    b_vecs = [gb_v[1, pl.ds(k * 16, 16)] for k in range(NV)]
    lanes = lax.iota(jnp.int32, 16)

    def gather(g, b):
        pltpu.async_copy(
            tok_hbm.at[idx_v.at[pl.ds(g * CHUNK, CHUNK)]], in_bufs[b],
            gsems[b])

    def gather_wait(b):
        pltpu.make_async_copy(
            tok_hbm.at[idx_v.at[pl.ds(0, CHUNK)]], in_bufs[b],
            gsems[b]).wait()

    def store_wait():
        pltpu.make_async_copy(
            out_hbm.at[pl.ds(base, CHUNK)], out_v, ssem).wait()

    gather(0, 0)
    gather(1, 1)

    @pl.loop(0, NCHUNK, step=2)
    def chunk_loop(g0):
        for b in range(2):
            g = g0 + b
            gather_wait(b)

            @pl.when(g >= 1)
            def _():
                store_wait()

            in_v = in_bufs[b]

            @pl.loop(0, SEQ)
            def pos_loop(p):
                pvs = [pos_v[p, pl.ds(k * 16, 16)] for k in range(NV)]
                for s in range(CHUNK // SEQ):
                    j = s * SEQ + p
                    vs = [in_v[j, pl.ds(k * 16, 16)] + pvs[k]
                          for k in range(NV)]
                    mean = _hsum(_tree_sum(vs), lanes) * (1.0 / D)
                    cs = [v - mean for v in vs]
                    var = _hsum(_tree_sum([c * c for c in cs]), lanes) * (1.0 / D)
                    r = _vrsqrt(var + 1e-5)
                    for k in range(NV):
                        out_v[j, pl.ds(k * 16, 16)] = (
                            cs[k] * (r * g_vecs[k]) + b_vecs[k])

            @pl.when(g + 2 < NCHUNK)
            def _():
                gather(g + 2, b)

            pltpu.async_copy(
                out_v, out_hbm.at[pl.ds(base + g * CHUNK, CHUNK)], ssem)

    store_wait()


@jax.jit
def kernel(x, tok_table, pos_table, gamma, beta):
    x_flat = x.reshape(-1).astype(jnp.int32)
    mesh = plsc.VectorSubcoreMesh(core_axis_name="c", subcore_axis_name="s")
    out = pl.kernel(
        _sc_body,
        out_type=jax.ShapeDtypeStruct((ROWS, D), jnp.float32),
        mesh=mesh,
        compiler_params=pltpu.CompilerParams(use_tc_tiling_on_sc=False),
        scratch_types=[
            pltpu.VMEM((ROWS_PER_W,), jnp.int32),
            pltpu.VMEM((CHUNK, D), jnp.float32),
            pltpu.VMEM((CHUNK, D), jnp.float32),
            pltpu.VMEM((CHUNK, D), jnp.float32),
            pltpu.VMEM((MAXLEN, D), jnp.float32),
            pltpu.VMEM((2, D), jnp.float32),
            pltpu.SemaphoreType.DMA,
            pltpu.SemaphoreType.DMA,
            pltpu.SemaphoreType.DMA,
        ],
    )(x_flat, tok_table, pos_table, gamma, beta)
    return out.reshape(BATCH, SEQ, D)
